# Initial kernel scaffold; baseline (speedup 1.0000x reference)
import jax
import jax.numpy as jnp
from jax.experimental import pallas as pl


def kernel(unknown, known, points1, points2, W1, b1, g1, be1, W2, b2, g2, be2):
    raise NotImplementedError("write your pallas kernel here")


# speedup vs baseline: 19.4836x; 19.4836x over previous
import functools

import jax
import jax.numpy as jnp
from jax import lax
from jax.experimental import pallas as pl
from jax.experimental.pallas import tpu as pltpu
from jax.experimental.pallas import tpu_sc as plsc

_SC_CORES = 2
_SC_SUBCORES = 16
_SC_WORKERS = _SC_CORES * _SC_SUBCORES

_TN1 = 256
_TNP = 512
_CHP = 32


def _topk3_body(un_ref, ut_ref, kt_ref, idx_ref, w_ref):
    b = pl.program_id(1)
    ut = ut_ref[0]
    kt = kt_ref[0]
    unk = un_ref[0]
    m = kt.shape[1]
    tn = ut.shape[1]
    dt = lax.dot_general(ut, kt, (((0,), (0,)), ((), ())),
                         preferred_element_type=jnp.float32)
    un_col = jnp.sum(unk * unk, axis=1, keepdims=True)
    kn_row = jnp.sum(kt * kt, axis=0, keepdims=True)
    dist = -2.0 * dt
    dist = dist + un_col
    dist = dist + kn_row
    iota = lax.broadcasted_iota(jnp.int32, (tn, m), 1)
    d = dist
    dks = []
    iks = []
    for k in range(3):
        mv = jnp.min(d, axis=1, keepdims=True)
        mi = jnp.min(jnp.where(d == mv, iota, m), axis=1, keepdims=True)
        dks.append(mv)
        iks.append(mi)
        if k < 2:
            d = jnp.where(iota == mi, jnp.float32(jnp.inf), d)
    recip = [1.0 / (dv + 1e-8) for dv in dks]
    norm = recip[0] + recip[1] + recip[2]
    w_ref[0] = jnp.concatenate([r / norm for r in recip], axis=1)
    idx_ref[0] = jnp.concatenate(iks, axis=1) + b * m


def _topk3(unknown, unknown_t, known_t):
    bsz, _, n = unknown_t.shape
    m = known_t.shape[2]
    grid = (n // _TN1, bsz)
    out_shapes = (
        jax.ShapeDtypeStruct((bsz, n, 3), jnp.int32),
        jax.ShapeDtypeStruct((bsz, n, 3), jnp.float32),
    )
    return pl.pallas_call(
        _topk3_body,
        grid=grid,
        in_specs=[
            pl.BlockSpec((1, _TN1, 3), lambda t, b: (b, t, 0)),
            pl.BlockSpec((1, 3, _TN1), lambda t, b: (b, 0, t)),
            pl.BlockSpec((1, 3, m), lambda t, b: (b, 0, 0)),
        ],
        out_specs=(
            pl.BlockSpec((1, _TN1, 3), lambda t, b: (b, t, 0)),
            pl.BlockSpec((1, _TN1, 3), lambda t, b: (b, t, 0)),
        ),
        out_shape=out_shapes,
    )(unknown, unknown_t, known_t)


def _interp_sc(p2rows, idxf, wsp):
    total_rows, c2 = p2rows.shape
    bn = idxf.shape[0] // 3
    pw = bn // _SC_WORKERS
    nit = pw // _CHP
    rpc = 3 * _CHP

    mesh = plsc.VectorSubcoreMesh(core_axis_name="c", subcore_axis_name="s")

    @functools.partial(
        pl.kernel,
        mesh=mesh,
        out_type=jax.ShapeDtypeStruct((bn, c2), jnp.float32),
        scratch_types=[
            pltpu.VMEM((rpc,), jnp.int32),
            pltpu.VMEM((rpc, 16), jnp.float32),
            pltpu.VMEM((rpc, c2), jnp.float32),
            pltpu.VMEM((_CHP, c2), jnp.float32),
            pltpu.SemaphoreType.DMA,
        ],
    )
    def sc_kernel(p2_hbm, idx_hbm, w_hbm, out_hbm, idx_v, w_v, rows_v, ob, sem):
        wid = lax.axis_index("s") * _SC_CORES + lax.axis_index("c")
        base_p = wid * pw

        def chunk(it, carry):
            p0 = base_p + it * _CHP
            pltpu.sync_copy(idx_hbm.at[pl.ds(p0 * 3, rpc)], idx_v)
            pltpu.sync_copy(w_hbm.at[pl.ds(p0 * 3, rpc)], w_v)
            pltpu.async_copy(p2_hbm.at[idx_v], rows_v, sem).wait()

            def pbody(p, c2_):
                r = 3 * p
                w0 = w_v[r, :]
                w1 = w_v[r + 1, :]
                w2 = w_v[r + 2, :]
                for cc in range(c2 // 16):
                    s = pl.ds(cc * 16, 16)
                    ob[p, s] = (rows_v[r, s] * w0 + rows_v[r + 1, s] * w1
                                + rows_v[r + 2, s] * w2)
                return c2_

            lax.fori_loop(0, _CHP, pbody, 0)
            pltpu.sync_copy(ob, out_hbm.at[pl.ds(p0, _CHP)])
            return carry

        lax.fori_loop(0, nit, chunk, 0)

    return sc_kernel(p2rows, idxf, wsp)


def _mlp1_body(p1_ref, xi_ref, w1a_ref, w1b_ref, b1_ref, pre_ref, st_ref):
    first = (pl.program_id(0) == 0) & (pl.program_id(1) == 0)
    xa = p1_ref[0]
    xb = xi_ref[0]
    y = lax.dot_general(xa, w1a_ref[...], (((0,), (1,)), ((), ())),
                        preferred_element_type=jnp.float32)
    y = y + lax.dot_general(xb, w1b_ref[...], (((1,), (1,)), ((), ())),
                            preferred_element_type=jnp.float32)
    y = y + b1_ref[...]
    s = jnp.sum(y, axis=0, keepdims=True)
    q = jnp.sum(y * y, axis=0, keepdims=True)
    sq = jnp.concatenate([s, q], axis=0)
    pre_ref[0] = y

    @pl.when(first)
    def _():
        st_ref[...] = sq

    @pl.when(jnp.logical_not(first))
    def _():
        st_ref[...] = st_ref[...] + sq


def _mlp2_body(x_ref, w2_ref, b2_ref, sc_ref, sh_ref, pre_ref, st_ref):
    first = (pl.program_id(0) == 0) & (pl.program_id(1) == 0)
    x = x_ref[0]
    h = jnp.maximum(x * sc_ref[...] + sh_ref[...], 0.0)
    y = lax.dot_general(w2_ref[...], h, (((1,), (1,)), ((), ())),
                        preferred_element_type=jnp.float32)
    y = y + b2_ref[...]
    s = jnp.sum(y, axis=1, keepdims=True)
    q = jnp.sum(y * y, axis=1, keepdims=True)
    sq = jnp.concatenate([s, q], axis=1)
    pre_ref[0] = y

    @pl.when(first)
    def _():
        st_ref[...] = sq

    @pl.when(jnp.logical_not(first))
    def _():
        st_ref[...] = st_ref[...] + sq


def _bnrelu_body(x_ref, sc_ref, sh_ref, out_ref):
    out_ref[0] = jnp.maximum(x_ref[0] * sc_ref[...] + sh_ref[...], 0.0)


def kernel(unknown, known, points1, points2, W1, b1, g1, be1, W2, b2, g2, be2):
    bsz, n, _ = unknown.shape
    m = known.shape[1]
    c1 = points1.shape[1]
    c2 = points2.shape[1]
    o1 = W1.shape[0]
    o2 = W2.shape[0]
    bn = bsz * n

    unknown_t = jnp.transpose(unknown, (0, 2, 1))
    known_t = jnp.transpose(known, (0, 2, 1))
    idx3, w3 = _topk3(unknown, unknown_t, known_t)

    p2rows = jnp.transpose(points2, (0, 2, 1)).reshape(bsz * m, c2)
    idxf = idx3.reshape(bn * 3)
    wsp = jnp.broadcast_to(w3.reshape(bn * 3)[:, None], (bn * 3, 16))
    interp = _interp_sc(p2rows, idxf, wsp)
    interp3 = interp.reshape(bsz, n, c2)

    w1a = W1[:, :c1]
    w1b = W1[:, c1:]
    grid = (bsz, n // _TNP)
    pre1, st1 = pl.pallas_call(
        _mlp1_body,
        grid=grid,
        in_specs=[
            pl.BlockSpec((1, c1, _TNP), lambda b, t: (b, 0, t)),
            pl.BlockSpec((1, _TNP, c2), lambda b, t: (b, t, 0)),
            pl.BlockSpec((o1, c1), lambda b, t: (0, 0)),
            pl.BlockSpec((o1, c2), lambda b, t: (0, 0)),
            pl.BlockSpec((1, o1), lambda b, t: (0, 0)),
        ],
        out_specs=(
            pl.BlockSpec((1, _TNP, o1), lambda b, t: (b, t, 0)),
            pl.BlockSpec((2, o1), lambda b, t: (0, 0)),
        ),
        out_shape=(
            jax.ShapeDtypeStruct((bsz, n, o1), jnp.float32),
            jax.ShapeDtypeStruct((2, o1), jnp.float32),
        ),
    )(points1, interp3, w1a, w1b, b1[None, :])

    mean1 = st1[0:1] / bn
    var1 = st1[1:2] / bn - mean1 * mean1
    sc1 = g1[None, :] / jnp.sqrt(var1 + 1e-5)
    sh1 = be1[None, :] - mean1 * sc1

    pre2, st2 = pl.pallas_call(
        _mlp2_body,
        grid=grid,
        in_specs=[
            pl.BlockSpec((1, _TNP, o1), lambda b, t: (b, t, 0)),
            pl.BlockSpec((o2, o1), lambda b, t: (0, 0)),
            pl.BlockSpec((o2, 1), lambda b, t: (0, 0)),
            pl.BlockSpec((1, o1), lambda b, t: (0, 0)),
            pl.BlockSpec((1, o1), lambda b, t: (0, 0)),
        ],
        out_specs=(
            pl.BlockSpec((1, o2, _TNP), lambda b, t: (b, 0, t)),
            pl.BlockSpec((o2, 2), lambda b, t: (0, 0)),
        ),
        out_shape=(
            jax.ShapeDtypeStruct((bsz, o2, n), jnp.float32),
            jax.ShapeDtypeStruct((o2, 2), jnp.float32),
        ),
    )(pre1, W2, b2[:, None], sc1, sh1)

    mean2 = st2[:, 0:1] / bn
    var2 = st2[:, 1:2] / bn - mean2 * mean2
    sc2 = g2[:, None] / jnp.sqrt(var2 + 1e-5)
    sh2 = be2[:, None] - mean2 * sc2

    out = pl.pallas_call(
        _bnrelu_body,
        grid=grid,
        in_specs=[
            pl.BlockSpec((1, o2, _TNP), lambda b, t: (b, 0, t)),
            pl.BlockSpec((o2, 1), lambda b, t: (0, 0)),
            pl.BlockSpec((o2, 1), lambda b, t: (0, 0)),
        ],
        out_specs=pl.BlockSpec((1, o2, _TNP), lambda b, t: (b, 0, t)),
        out_shape=jax.ShapeDtypeStruct((bsz, o2, n), jnp.float32),
    )(pre2, sc2, sh2)
    return out


# speedup vs baseline: 22.4835x; 1.1540x over previous
import functools

import jax
import jax.numpy as jnp
from jax import lax
from jax.experimental import pallas as pl
from jax.experimental.pallas import tpu as pltpu
from jax.experimental.pallas import tpu_sc as plsc

_SC_CORES = 2
_SC_SUBCORES = 16
_SC_WORKERS = _SC_CORES * _SC_SUBCORES

_TN1 = 256
_TNP = 512
_CHP = 32


def _topk3_body(un_ref, ut_ref, kt_ref, idx_ref, w_ref):
    b = pl.program_id(1)
    ut = ut_ref[0]
    kt = kt_ref[0]
    unk = un_ref[0]
    m = kt.shape[1]
    tn = ut.shape[1]
    dt = lax.dot_general(ut, kt, (((0,), (0,)), ((), ())),
                         preferred_element_type=jnp.float32)
    un_col = jnp.sum(unk * unk, axis=1, keepdims=True)
    kn_row = jnp.sum(kt * kt, axis=0, keepdims=True)
    dist = -2.0 * dt
    dist = dist + un_col
    dist = dist + kn_row
    iota = lax.broadcasted_iota(jnp.int32, (tn, m), 1).astype(jnp.float32)
    d = dist
    dks = []
    iks = []
    for k in range(3):
        mv = jnp.min(d, axis=1, keepdims=True)
        mi = jnp.min(jnp.where(d == mv, iota, jnp.float32(m)),
                     axis=1, keepdims=True)
        dks.append(mv)
        iks.append(mi)
        if k < 2:
            d = jnp.where(iota == mi, jnp.float32(jnp.inf), d)
    recip = [1.0 / (dv + 1e-8) for dv in dks]
    norm = recip[0] + recip[1] + recip[2]
    w_ref[0] = jnp.concatenate([r / norm for r in recip], axis=1)
    idx_ref[0] = (jnp.concatenate(iks, axis=1).astype(jnp.int32)
                  + b * m)


def _topk3(unknown, unknown_t, known_t):
    bsz, _, n = unknown_t.shape
    m = known_t.shape[2]
    grid = (n // _TN1, bsz)
    out_shapes = (
        jax.ShapeDtypeStruct((bsz, n, 3), jnp.int32),
        jax.ShapeDtypeStruct((bsz, n, 3), jnp.float32),
    )
    return pl.pallas_call(
        _topk3_body,
        grid=grid,
        in_specs=[
            pl.BlockSpec((1, _TN1, 3), lambda t, b: (b, t, 0)),
            pl.BlockSpec((1, 3, _TN1), lambda t, b: (b, 0, t)),
            pl.BlockSpec((1, 3, m), lambda t, b: (b, 0, 0)),
        ],
        out_specs=(
            pl.BlockSpec((1, _TN1, 3), lambda t, b: (b, t, 0)),
            pl.BlockSpec((1, _TN1, 3), lambda t, b: (b, t, 0)),
        ),
        out_shape=out_shapes,
    )(unknown, unknown_t, known_t)


def _interp_sc(p2rows, idxf, wsp):
    total_rows, c2 = p2rows.shape
    bn = idxf.shape[0] // 3
    pw = bn // _SC_WORKERS
    nit = pw // _CHP
    rpc = 3 * _CHP

    mesh = plsc.VectorSubcoreMesh(core_axis_name="c", subcore_axis_name="s")

    @functools.partial(
        pl.kernel,
        mesh=mesh,
        out_type=jax.ShapeDtypeStruct((bn, c2), jnp.float32),
        scratch_types=[
            pltpu.VMEM((rpc,), jnp.int32),
            pltpu.VMEM((rpc,), jnp.int32),
            pltpu.VMEM((rpc, 16), jnp.float32),
            pltpu.VMEM((rpc, 16), jnp.float32),
            pltpu.VMEM((rpc, c2), jnp.float32),
            pltpu.VMEM((rpc, c2), jnp.float32),
            pltpu.VMEM((_CHP, c2), jnp.float32),
            pltpu.VMEM((_CHP, c2), jnp.float32),
            pltpu.SemaphoreType.DMA, pltpu.SemaphoreType.DMA,
            pltpu.SemaphoreType.DMA, pltpu.SemaphoreType.DMA,
            pltpu.SemaphoreType.DMA, pltpu.SemaphoreType.DMA,
        ],
    )
    def sc_kernel(p2_hbm, idx_hbm, w_hbm, out_hbm,
                  i0, i1, w0, w1, r0, r1, o0, o1,
                  sg0, sg1, so0, so1, si0, si1):
        wid = lax.axis_index("s") * _SC_CORES + lax.axis_index("c")
        base_p = wid * pw
        bufs = ((i0, w0, r0, o0, sg0, so0, si0),
                (i1, w1, r1, o1, sg1, so1, si1))

        def fetch_idx(it, slot):
            iv, wv, _, _, _, _, si = bufs[slot]
            p0 = base_p + it * _CHP
            pltpu.async_copy(idx_hbm.at[pl.ds(p0 * 3, rpc)], iv, si)
            pltpu.async_copy(w_hbm.at[pl.ds(p0 * 3, rpc)], wv, si)

        def wait_idx(slot):
            iv, wv, _, _, _, _, si = bufs[slot]
            pltpu.make_async_copy(idx_hbm.at[pl.ds(0, rpc)], iv, si).wait()
            pltpu.make_async_copy(w_hbm.at[pl.ds(0, rpc)], wv, si).wait()

        def launch_gather(slot):
            iv, _, rv, _, sg, _, _ = bufs[slot]
            pltpu.async_copy(p2_hbm.at[iv], rv, sg)

        fetch_idx(0, 0)
        fetch_idx(1, 1)
        wait_idx(0)
        launch_gather(0)

        def step(it, slot):
            iv, wv, rv, ov, sg, so, si = bufs[slot]
            p0 = base_p + it * _CHP
            pltpu.make_async_copy(p2_hbm.at[iv], rv, sg).wait()

            @pl.when(it >= 2)
            def _():
                pltpu.make_async_copy(ov, out_hbm.at[pl.ds(0, _CHP)], so).wait()

            def pbody(p, carry):
                r = 3 * p
                a0 = wv[r, :]
                a1 = wv[r + 1, :]
                a2 = wv[r + 2, :]
                for cc in range(c2 // 16):
                    s = pl.ds(cc * 16, 16)
                    ov[p, s] = (rv[r, s] * a0 + rv[r + 1, s] * a1
                                + rv[r + 2, s] * a2)
                return carry

            lax.fori_loop(0, _CHP, pbody, 0, unroll=4)
            pltpu.async_copy(ov, out_hbm.at[pl.ds(p0, _CHP)], so)

            @pl.when(it + 2 < nit)
            def _():
                fetch_idx(it + 2, slot)

            @pl.when(it + 1 < nit)
            def _():
                wait_idx(slot ^ 1)
                launch_gather(slot ^ 1)

        def pair(ii, carry):
            it = 2 * ii
            step(it, 0)
            step(it + 1, 1)
            return carry

        lax.fori_loop(0, nit // 2, pair, 0)
        pltpu.make_async_copy(o0, out_hbm.at[pl.ds(0, _CHP)], so0).wait()
        pltpu.make_async_copy(o1, out_hbm.at[pl.ds(0, _CHP)], so1).wait()

    return sc_kernel(p2rows, idxf, wsp)


def _mlp1_body(p1_ref, xi_ref, w1a_ref, w1b_ref, b1_ref, pre_ref, st_ref):
    first = (pl.program_id(0) == 0) & (pl.program_id(1) == 0)
    xa = p1_ref[0]
    xb = xi_ref[0]
    y = lax.dot_general(xa, w1a_ref[...], (((0,), (1,)), ((), ())),
                        preferred_element_type=jnp.float32)
    y = y + lax.dot_general(xb, w1b_ref[...], (((1,), (1,)), ((), ())),
                            preferred_element_type=jnp.float32)
    y = y + b1_ref[...]
    s = jnp.sum(y, axis=0, keepdims=True)
    q = jnp.sum(y * y, axis=0, keepdims=True)
    sq = jnp.concatenate([s, q], axis=0)
    pre_ref[0] = y

    @pl.when(first)
    def _():
        st_ref[...] = sq

    @pl.when(jnp.logical_not(first))
    def _():
        st_ref[...] = st_ref[...] + sq


def _mlp2_body(x_ref, w2_ref, b2_ref, sc_ref, sh_ref, pre_ref, st_ref):
    first = (pl.program_id(0) == 0) & (pl.program_id(1) == 0)
    x = x_ref[0]
    h = jnp.maximum(x * sc_ref[...] + sh_ref[...], 0.0)
    y = lax.dot_general(w2_ref[...], h, (((1,), (1,)), ((), ())),
                        preferred_element_type=jnp.float32)
    y = y + b2_ref[...]
    s = jnp.sum(y, axis=1, keepdims=True)
    q = jnp.sum(y * y, axis=1, keepdims=True)
    sq = jnp.concatenate([s, q], axis=1)
    pre_ref[0] = y

    @pl.when(first)
    def _():
        st_ref[...] = sq

    @pl.when(jnp.logical_not(first))
    def _():
        st_ref[...] = st_ref[...] + sq


def _bnrelu_body(x_ref, sc_ref, sh_ref, out_ref):
    out_ref[0] = jnp.maximum(x_ref[0] * sc_ref[...] + sh_ref[...], 0.0)


def kernel(unknown, known, points1, points2, W1, b1, g1, be1, W2, b2, g2, be2):
    bsz, n, _ = unknown.shape
    m = known.shape[1]
    c1 = points1.shape[1]
    c2 = points2.shape[1]
    o1 = W1.shape[0]
    o2 = W2.shape[0]
    bn = bsz * n

    unknown_t = jnp.transpose(unknown, (0, 2, 1))
    known_t = jnp.transpose(known, (0, 2, 1))
    idx3, w3 = _topk3(unknown, unknown_t, known_t)

    p2rows = jnp.transpose(points2, (0, 2, 1)).reshape(bsz * m, c2)
    idxf = idx3.reshape(bn * 3)
    wsp = jnp.broadcast_to(w3.reshape(bn * 3)[:, None], (bn * 3, 16))
    interp = _interp_sc(p2rows, idxf, wsp)
    interp3 = interp.reshape(bsz, n, c2)

    w1a = W1[:, :c1]
    w1b = W1[:, c1:]
    grid = (bsz, n // _TNP)
    pre1, st1 = pl.pallas_call(
        _mlp1_body,
        grid=grid,
        in_specs=[
            pl.BlockSpec((1, c1, _TNP), lambda b, t: (b, 0, t)),
            pl.BlockSpec((1, _TNP, c2), lambda b, t: (b, t, 0)),
            pl.BlockSpec((o1, c1), lambda b, t: (0, 0)),
            pl.BlockSpec((o1, c2), lambda b, t: (0, 0)),
            pl.BlockSpec((1, o1), lambda b, t: (0, 0)),
        ],
        out_specs=(
            pl.BlockSpec((1, _TNP, o1), lambda b, t: (b, t, 0)),
            pl.BlockSpec((2, o1), lambda b, t: (0, 0)),
        ),
        out_shape=(
            jax.ShapeDtypeStruct((bsz, n, o1), jnp.float32),
            jax.ShapeDtypeStruct((2, o1), jnp.float32),
        ),
    )(points1, interp3, w1a, w1b, b1[None, :])

    mean1 = st1[0:1] / bn
    var1 = st1[1:2] / bn - mean1 * mean1
    sc1 = g1[None, :] / jnp.sqrt(var1 + 1e-5)
    sh1 = be1[None, :] - mean1 * sc1

    pre2, st2 = pl.pallas_call(
        _mlp2_body,
        grid=grid,
        in_specs=[
            pl.BlockSpec((1, _TNP, o1), lambda b, t: (b, t, 0)),
            pl.BlockSpec((o2, o1), lambda b, t: (0, 0)),
            pl.BlockSpec((o2, 1), lambda b, t: (0, 0)),
            pl.BlockSpec((1, o1), lambda b, t: (0, 0)),
            pl.BlockSpec((1, o1), lambda b, t: (0, 0)),
        ],
        out_specs=(
            pl.BlockSpec((1, o2, _TNP), lambda b, t: (b, 0, t)),
            pl.BlockSpec((o2, 2), lambda b, t: (0, 0)),
        ),
        out_shape=(
            jax.ShapeDtypeStruct((bsz, o2, n), jnp.float32),
            jax.ShapeDtypeStruct((o2, 2), jnp.float32),
        ),
    )(pre1, W2, b2[:, None], sc1, sh1)

    mean2 = st2[:, 0:1] / bn
    var2 = st2[:, 1:2] / bn - mean2 * mean2
    sc2 = g2[:, None] / jnp.sqrt(var2 + 1e-5)
    sh2 = be2[:, None] - mean2 * sc2

    out = pl.pallas_call(
        _bnrelu_body,
        grid=grid,
        in_specs=[
            pl.BlockSpec((1, o2, _TNP), lambda b, t: (b, 0, t)),
            pl.BlockSpec((o2, 1), lambda b, t: (0, 0)),
            pl.BlockSpec((o2, 1), lambda b, t: (0, 0)),
        ],
        out_specs=pl.BlockSpec((1, o2, _TNP), lambda b, t: (b, 0, t)),
        out_shape=jax.ShapeDtypeStruct((bsz, o2, n), jnp.float32),
    )(pre2, sc2, sh2)
    return out


# speedup vs baseline: 26.3033x; 1.1699x over previous
import functools

import jax
import jax.numpy as jnp
from jax import lax
from jax.experimental import pallas as pl
from jax.experimental.pallas import tpu as pltpu
from jax.experimental.pallas import tpu_sc as plsc

_SC_CORES = 2
_SC_SUBCORES = 16
_SC_WORKERS = _SC_CORES * _SC_SUBCORES

_TN1 = 256
_TNP = 512
_CHP = 32


def _topk3_body(un_ref, ut_ref, kt_ref, idx_ref, w_ref):
    b = pl.program_id(1)
    ut = ut_ref[0]
    kt = kt_ref[0]
    unk = un_ref[0]
    m = kt.shape[1]
    tn = ut.shape[1]
    dt = lax.dot_general(ut, kt, (((0,), (0,)), ((), ())),
                         preferred_element_type=jnp.float32)
    usq = unk * unk
    un_col = (usq[:, 0:1] + usq[:, 1:2]) + usq[:, 2:3]
    ksq = kt * kt
    kn_row = (ksq[0:1, :] + ksq[1:2, :]) + ksq[2:3, :]
    dist = -2.0 * dt
    dist = dist + un_col
    dist = dist + kn_row
    iota = lax.broadcasted_iota(jnp.int32, (tn, m), 1).astype(jnp.float32)
    d = dist
    dks = []
    iks = []
    for k in range(3):
        mv = jnp.min(d, axis=1, keepdims=True)
        mi = jnp.min(jnp.where(d == mv, iota, jnp.float32(m)),
                     axis=1, keepdims=True)
        dks.append(mv)
        iks.append(mi)
        if k < 2:
            d = jnp.where(iota == mi, jnp.float32(jnp.inf), d)
    recip = [1.0 / (dv + 1e-8) for dv in dks]
    norm = recip[0] + recip[1] + recip[2]
    w_ref[0] = jnp.concatenate(
        [jnp.broadcast_to(r / norm, (tn, 16)) for r in recip], axis=1)
    idx_ref[0] = (jnp.concatenate(iks, axis=1).astype(jnp.int32)
                  + b * m)


def _topk3(unknown, unknown_t, known_t):
    bsz, _, n = unknown_t.shape
    m = known_t.shape[2]
    grid = (n // _TN1, bsz)
    out_shapes = (
        jax.ShapeDtypeStruct((bsz, n, 3), jnp.int32),
        jax.ShapeDtypeStruct((bsz, n, 48), jnp.float32),
    )
    return pl.pallas_call(
        _topk3_body,
        grid=grid,
        in_specs=[
            pl.BlockSpec((1, _TN1, 3), lambda t, b: (b, t, 0)),
            pl.BlockSpec((1, 3, _TN1), lambda t, b: (b, 0, t)),
            pl.BlockSpec((1, 3, m), lambda t, b: (b, 0, 0)),
        ],
        out_specs=(
            pl.BlockSpec((1, _TN1, 3), lambda t, b: (b, t, 0)),
            pl.BlockSpec((1, _TN1, 48), lambda t, b: (b, t, 0)),
        ),
        out_shape=out_shapes,
    )(unknown, unknown_t, known_t)


def _interp_sc(p2rows, idxf, wsp):
    total_rows, c2 = p2rows.shape
    bn = idxf.shape[0] // 3
    pw = bn // _SC_WORKERS
    nit = pw // _CHP
    rpc = 3 * _CHP

    mesh = plsc.VectorSubcoreMesh(core_axis_name="c", subcore_axis_name="s")

    @functools.partial(
        pl.kernel,
        mesh=mesh,
        out_type=jax.ShapeDtypeStruct((bn, c2), jnp.float32),
        scratch_types=[
            pltpu.VMEM((rpc,), jnp.int32),
            pltpu.VMEM((rpc,), jnp.int32),
            pltpu.VMEM((rpc, 16), jnp.float32),
            pltpu.VMEM((rpc, 16), jnp.float32),
            pltpu.VMEM((rpc, c2), jnp.float32),
            pltpu.VMEM((rpc, c2), jnp.float32),
            pltpu.VMEM((_CHP, c2), jnp.float32),
            pltpu.VMEM((_CHP, c2), jnp.float32),
            pltpu.SemaphoreType.DMA, pltpu.SemaphoreType.DMA,
            pltpu.SemaphoreType.DMA, pltpu.SemaphoreType.DMA,
            pltpu.SemaphoreType.DMA, pltpu.SemaphoreType.DMA,
        ],
    )
    def sc_kernel(p2_hbm, idx_hbm, w_hbm, out_hbm,
                  i0, i1, w0, w1, r0, r1, o0, o1,
                  sg0, sg1, so0, so1, si0, si1):
        wid = lax.axis_index("s") * _SC_CORES + lax.axis_index("c")
        base_p = wid * pw
        bufs = ((i0, w0, r0, o0, sg0, so0, si0),
                (i1, w1, r1, o1, sg1, so1, si1))

        def fetch_idx(it, slot):
            iv, wv, _, _, _, _, si = bufs[slot]
            p0 = base_p + it * _CHP
            pltpu.async_copy(idx_hbm.at[pl.ds(p0 * 3, rpc)], iv, si)
            pltpu.async_copy(w_hbm.at[pl.ds(p0 * 3, rpc)], wv, si)

        def wait_idx(slot):
            iv, wv, _, _, _, _, si = bufs[slot]
            pltpu.make_async_copy(idx_hbm.at[pl.ds(0, rpc)], iv, si).wait()
            pltpu.make_async_copy(w_hbm.at[pl.ds(0, rpc)], wv, si).wait()

        def launch_gather(slot):
            iv, _, rv, _, sg, _, _ = bufs[slot]
            pltpu.async_copy(p2_hbm.at[iv], rv, sg)

        fetch_idx(0, 0)
        fetch_idx(1, 1)
        wait_idx(0)
        launch_gather(0)

        def step(it, slot):
            iv, wv, rv, ov, sg, so, si = bufs[slot]
            p0 = base_p + it * _CHP
            pltpu.make_async_copy(p2_hbm.at[iv], rv, sg).wait()

            @pl.when(it >= 2)
            def _():
                pltpu.make_async_copy(ov, out_hbm.at[pl.ds(0, _CHP)], so).wait()

            def pbody(p, carry):
                r = 3 * p
                a0 = wv[r, :]
                a1 = wv[r + 1, :]
                a2 = wv[r + 2, :]
                for cc in range(c2 // 16):
                    s = pl.ds(cc * 16, 16)
                    ov[p, s] = (rv[r, s] * a0 + rv[r + 1, s] * a1
                                + rv[r + 2, s] * a2)
                return carry

            lax.fori_loop(0, _CHP, pbody, 0, unroll=4)
            pltpu.async_copy(ov, out_hbm.at[pl.ds(p0, _CHP)], so)

            @pl.when(it + 2 < nit)
            def _():
                fetch_idx(it + 2, slot)

            @pl.when(it + 1 < nit)
            def _():
                wait_idx(slot ^ 1)
                launch_gather(slot ^ 1)

        def pair(ii, carry):
            it = 2 * ii
            step(it, 0)
            step(it + 1, 1)
            return carry

        lax.fori_loop(0, nit // 2, pair, 0)
        pltpu.make_async_copy(o0, out_hbm.at[pl.ds(0, _CHP)], so0).wait()
        pltpu.make_async_copy(o1, out_hbm.at[pl.ds(0, _CHP)], so1).wait()

    return sc_kernel(p2rows, idxf, wsp)


def _mlp1_body(p1_ref, xi_ref, w1a_ref, w1b_ref, b1_ref, pre_ref, st_ref):
    first = (pl.program_id(0) == 0) & (pl.program_id(1) == 0)
    xa = p1_ref[0]
    xb = xi_ref[0]
    y = lax.dot_general(xa, w1a_ref[...], (((0,), (1,)), ((), ())),
                        preferred_element_type=jnp.float32)
    y = y + lax.dot_general(xb, w1b_ref[...], (((1,), (1,)), ((), ())),
                            preferred_element_type=jnp.float32)
    y = y + b1_ref[...]
    s = jnp.sum(y, axis=0, keepdims=True)
    q = jnp.sum(y * y, axis=0, keepdims=True)
    sq = jnp.concatenate([s, q], axis=0)
    pre_ref[0] = y

    @pl.when(first)
    def _():
        st_ref[...] = sq

    @pl.when(jnp.logical_not(first))
    def _():
        st_ref[...] = st_ref[...] + sq


def _mlp2_body(x_ref, w2_ref, b2_ref, sc_ref, sh_ref, pre_ref, st_ref):
    first = (pl.program_id(0) == 0) & (pl.program_id(1) == 0)
    x = x_ref[0]
    h = jnp.maximum(x * sc_ref[...] + sh_ref[...], 0.0)
    y = lax.dot_general(w2_ref[...], h, (((1,), (1,)), ((), ())),
                        preferred_element_type=jnp.float32)
    y = y + b2_ref[...]
    s = jnp.sum(y, axis=1, keepdims=True)
    q = jnp.sum(y * y, axis=1, keepdims=True)
    sq = jnp.concatenate([s, q], axis=1)
    pre_ref[0] = y

    @pl.when(first)
    def _():
        st_ref[...] = sq

    @pl.when(jnp.logical_not(first))
    def _():
        st_ref[...] = st_ref[...] + sq


def _bnrelu_body(xa_ref, xb_ref, sc_ref, sh_ref, out_ref):
    half = pl.num_programs(0) // 2
    b = pl.program_id(0)

    @pl.when(b < half)
    def _():
        out_ref[0] = jnp.maximum(xa_ref[0] * sc_ref[...] + sh_ref[...], 0.0)

    @pl.when(b >= half)
    def _():
        out_ref[0] = jnp.maximum(xb_ref[0] * sc_ref[...] + sh_ref[...], 0.0)


def _mlp1(points1, interp3, w1a, w1b, b1):
    bsz, c1, n = points1.shape
    c2 = interp3.shape[2]
    o1 = w1a.shape[0]
    grid = (bsz, n // _TNP)
    return pl.pallas_call(
        _mlp1_body,
        grid=grid,
        in_specs=[
            pl.BlockSpec((1, c1, _TNP), lambda b, t: (b, 0, t)),
            pl.BlockSpec((1, _TNP, c2), lambda b, t: (b, t, 0)),
            pl.BlockSpec((o1, c1), lambda b, t: (0, 0)),
            pl.BlockSpec((o1, c2), lambda b, t: (0, 0)),
            pl.BlockSpec((1, o1), lambda b, t: (0, 0)),
        ],
        out_specs=(
            pl.BlockSpec((1, _TNP, o1), lambda b, t: (b, t, 0)),
            pl.BlockSpec((2, o1), lambda b, t: (0, 0)),
        ),
        out_shape=(
            jax.ShapeDtypeStruct((bsz, n, o1), jnp.float32),
            jax.ShapeDtypeStruct((2, o1), jnp.float32),
        ),
    )(points1, interp3, w1a, w1b, b1)


def _mlp2(pre1, W2, b2c, sc1, sh1):
    bsz, n, o1 = pre1.shape
    o2 = W2.shape[0]
    grid = (bsz, n // _TNP)
    return pl.pallas_call(
        _mlp2_body,
        grid=grid,
        in_specs=[
            pl.BlockSpec((1, _TNP, o1), lambda b, t: (b, t, 0)),
            pl.BlockSpec((o2, o1), lambda b, t: (0, 0)),
            pl.BlockSpec((o2, 1), lambda b, t: (0, 0)),
            pl.BlockSpec((1, o1), lambda b, t: (0, 0)),
            pl.BlockSpec((1, o1), lambda b, t: (0, 0)),
        ],
        out_specs=(
            pl.BlockSpec((1, o2, _TNP), lambda b, t: (b, 0, t)),
            pl.BlockSpec((o2, 2), lambda b, t: (0, 0)),
        ),
        out_shape=(
            jax.ShapeDtypeStruct((bsz, o2, n), jnp.float32),
            jax.ShapeDtypeStruct((o2, 2), jnp.float32),
        ),
    )(pre1, W2, b2c, sc1, sh1)


def kernel(unknown, known, points1, points2, W1, b1, g1, be1, W2, b2, g2, be2):
    bsz, n, _ = unknown.shape
    m = known.shape[1]
    c1 = points1.shape[1]
    c2 = points2.shape[1]
    o1 = W1.shape[0]
    o2 = W2.shape[0]
    bn = bsz * n
    w1a = W1[:, :c1]
    w1b = W1[:, c1:]

    nb2 = bsz // 2
    halves = []
    for h in range(2):
        unk = unknown[h * nb2:(h + 1) * nb2]
        kno = known[h * nb2:(h + 1) * nb2]
        p2h = points2[h * nb2:(h + 1) * nb2]
        idx3, w48 = _topk3(unk, jnp.transpose(unk, (0, 2, 1)),
                           jnp.transpose(kno, (0, 2, 1)))
        p2rows = jnp.transpose(p2h, (0, 2, 1)).reshape(nb2 * m, c2)
        idxf = idx3.reshape(nb2 * n * 3)
        wsp = w48.reshape(nb2 * n * 3, 16)
        interp3 = _interp_sc(p2rows, idxf, wsp).reshape(nb2, n, c2)
        halves.append(interp3)

    pre1s, st1s = [], []
    for h in range(2):
        pre1, st1 = _mlp1(points1[h * nb2:(h + 1) * nb2], halves[h],
                          w1a, w1b, b1[None, :])
        pre1s.append(pre1)
        st1s.append(st1)
    st1 = st1s[0] + st1s[1]
    mean1 = st1[0:1] / bn
    var1 = st1[1:2] / bn - mean1 * mean1
    sc1 = g1[None, :] / jnp.sqrt(var1 + 1e-5)
    sh1 = be1[None, :] - mean1 * sc1

    pre2s, st2s = [], []
    for h in range(2):
        pre2, st2 = _mlp2(pre1s[h], W2, b2[:, None], sc1, sh1)
        pre2s.append(pre2)
        st2s.append(st2)
    st2 = st2s[0] + st2s[1]
    mean2 = st2[:, 0:1] / bn
    var2 = st2[:, 1:2] / bn - mean2 * mean2
    sc2 = g2[:, None] / jnp.sqrt(var2 + 1e-5)
    sh2 = be2[:, None] - mean2 * sc2

    grid = (bsz, n // _TNP)
    out = pl.pallas_call(
        _bnrelu_body,
        grid=grid,
        in_specs=[
            pl.BlockSpec((1, o2, _TNP), lambda b, t: (b % (bsz // 2), 0, t)),
            pl.BlockSpec((1, o2, _TNP), lambda b, t: (b % (bsz // 2), 0, t)),
            pl.BlockSpec((o2, 1), lambda b, t: (0, 0)),
            pl.BlockSpec((o2, 1), lambda b, t: (0, 0)),
        ],
        out_specs=pl.BlockSpec((1, o2, _TNP), lambda b, t: (b, 0, t)),
        out_shape=jax.ShapeDtypeStruct((bsz, o2, n), jnp.float32),
    )(pre2s[0], pre2s[1], sc2, sh2)
    return out
